# baseline (speedup 1.0000x reference)
import jax
import jax.numpy as jnp
from jax.experimental import pallas as pl
from jax.experimental.pallas import tpu as pltpu
from jax.experimental.pallas import tpu_sc as plsc

_W = 128
_LANES = 16


def kernel(input_emb, position, pe):
    B, N, L, D = input_emb.shape
    R = B * N * L
    x = input_emb.reshape(R, D)
    idx = position.reshape(1, R).astype(jnp.int32)

    mesh = plsc.VectorSubcoreMesh(core_axis_name="c", subcore_axis_name="s")

    @jax.jit
    def run(x, idx, pe):
        @pl.kernel(out_type=jax.ShapeDtypeStruct((R, D), jnp.float32), mesh=mesh)
        def emb_add(x_hbm, i_hbm, pe_hbm, o_hbm):
            def body(i_vmem, x_vmem, o_vmem):
                pltpu.sync_copy(pe_hbm.at[i_vmem.at[0]], o_vmem)

                @pl.loop(0, _W)
                def _(r):
                    for c in range(0, D, _LANES):
                        plsc.addupdate(
                            o_vmem.at[r, pl.ds(c, _LANES)],
                            x_vmem.at[r, pl.ds(c, _LANES)][...],
                        )

            pltpu.emit_pipeline(
                body,
                grid=(R // _W,),
                in_specs=[
                    pl.BlockSpec((1, _W), lambda i: (0, i)),
                    pl.BlockSpec((_W, D), lambda i: (i, 0)),
                ],
                out_specs=[pl.BlockSpec((_W, D), lambda i: (i, 0))],
                core_axis_name=("c", "s"),
                dimension_semantics=(pltpu.PARALLEL,),
            )(i_hbm, x_hbm, o_hbm)

        return emb_add(x, idx, pe)

    return run(x, idx, pe).reshape(B, N, L, D)


# speedup vs baseline: 1.2450x; 1.2450x over previous
import jax
import jax.numpy as jnp
from jax.experimental import pallas as pl
from jax.experimental.pallas import tpu as pltpu
from jax.experimental.pallas import tpu_sc as plsc

_G = 10
_LANES = 16


def kernel(input_emb, position, pe):
    B, N, L, D = input_emb.shape
    BN = B * N
    x = input_emb.reshape(BN, L, D)
    steps = BN // _G
    idx = position.reshape(steps, _G * L).astype(jnp.int32)

    mesh = plsc.VectorSubcoreMesh(core_axis_name="c", subcore_axis_name="s")

    @jax.jit
    def run(x, idx, pe):
        @pl.kernel(
            out_type=jax.ShapeDtypeStruct((BN, L, D), jnp.float32),
            mesh=mesh,
            scratch_types=[pltpu.VMEM((_G * L, D), jnp.float32)],
        )
        def emb_add(x_hbm, i_hbm, pe_hbm, o_hbm, pe_rows):
            def body(i_vmem, x_vmem, o_vmem):
                pltpu.sync_copy(pe_hbm.at[i_vmem.at[0]], pe_rows)

                @pl.loop(0, _G)
                def _(g):
                    @pl.loop(0, L)
                    def _(l):
                        r = g * L + l
                        for c in range(0, D, _LANES):
                            o_vmem.at[g, l, pl.ds(c, _LANES)][...] = (
                                pe_rows.at[r, pl.ds(c, _LANES)][...]
                                + x_vmem.at[g, l, pl.ds(c, _LANES)][...]
                            )

            pltpu.emit_pipeline(
                body,
                grid=(steps,),
                in_specs=[
                    pl.BlockSpec((1, _G * L), lambda i: (i, 0)),
                    pl.BlockSpec((_G, L, D), lambda i: (i, 0, 0)),
                ],
                out_specs=[pl.BlockSpec((_G, L, D), lambda i: (i, 0, 0))],
                core_axis_name=("c", "s"),
                dimension_semantics=(pltpu.PARALLEL,),
            )(i_hbm, x_hbm, o_hbm)

        return emb_add(x, idx, pe)

    return run(x, idx, pe).reshape(B, N, L, D)
